# baseline (speedup 1.0000x reference)
import functools

import jax
import jax.numpy as jnp
from jax import lax
from jax.experimental import pallas as pl
from jax.experimental.pallas import tpu as pltpu
from jax.experimental.pallas import tpu_sc as plsc

V = 100000
D = 512
L = 16
NC = 2
NS = 16
NW = NC * NS
GROUPS = V // L
GQ = GROUPS // NW
GR = GROUPS - NW * GQ


def _sc_topk_call(psi_flat, map2d):
    mesh = plsc.VectorSubcoreMesh(core_axis_name="c", subcore_axis_name="s")

    @functools.partial(
        pl.kernel,
        mesh=mesh,
        out_type=[
            jax.ShapeDtypeStruct((NW, L), jnp.float32),
            jax.ShapeDtypeStruct((NW, L), jnp.float32),
            jax.ShapeDtypeStruct((NW, L), jnp.int32),
        ],
        scratch_types=[
            pltpu.VMEM((D,), jnp.float32),
            pltpu.VMEM((L, D), jnp.float32),
            pltpu.VMEM((L,), jnp.float32),
            pltpu.VMEM((L,), jnp.float32),
            pltpu.VMEM((L,), jnp.int32),
        ],
    )
    def k(psi_hbm, map_hbm, out_num, out_den, out_idx, psi_v, buf_v,
          rn_v, rd_v, ri_v):
        wid = lax.axis_index("c") * NS + lax.axis_index("s")
        n_blocks = jnp.where(wid < GR, GQ + 1, GQ)
        base_row = (wid * GQ + jnp.minimum(wid, GR)) * L

        pltpu.sync_copy(psi_hbm, psi_v)
        psi_regs = [psi_v[pl.ds(L * c, L)] for c in range(D // L)]
        zero = jnp.zeros((L,), jnp.float32)

        def ssum(vec):
            vals = [vec[i] for i in range(L)]
            while len(vals) > 1:
                vals = [vals[2 * i] + vals[2 * i + 1]
                        for i in range(len(vals) // 2)]
            return vals[0]

        def block(b, carry):
            bn, bd, bi = carry
            row0 = base_row + b * L
            pltpu.sync_copy(map_hbm.at[pl.ds(row0, L)], buf_v)
            for r in range(L):
                accn = zero
                accs = zero
                for c in range(D // L):
                    x = buf_v[r, pl.ds(L * c, L)]
                    accn = accn + x * psi_regs[c]
                    accs = accs + x * x
                num = ssum(accn)
                ssq = jnp.maximum(ssum(accs), jnp.float32(1e-16))
                n2 = num * jnp.where(num < 0.0, -num, num)
                upd = n2 * bd > bn * ssq
                bn = jnp.where(upd, n2, bn)
                bd = jnp.where(upd, ssq, bd)
                bi = jnp.where(upd, row0 + r, bi)
            return bn, bd, bi

        init = (jnp.float32(-3.4e38), jnp.float32(1.0), jnp.int32(0))
        bn, bd, bi = lax.fori_loop(0, n_blocks, block, init)
        rn_v[...] = jnp.full((L,), 1.0, jnp.float32) * bn
        rd_v[...] = jnp.full((L,), 1.0, jnp.float32) * bd
        ri_v[...] = jnp.full((L,), 1, jnp.int32) * bi
        pltpu.sync_copy(rn_v, out_num.at[wid])
        pltpu.sync_copy(rd_v, out_den.at[wid])
        pltpu.sync_copy(ri_v, out_idx.at[wid])

    return k(psi_flat, map2d)


def kernel(psi_final, spectral_map):
    psi_flat = psi_final.reshape(-1)
    map2d = spectral_map.reshape(V, D)
    nums, dens, idxs = _sc_topk_call(psi_flat, map2d)
    n = nums[:, 0]
    d = dens[:, 0]
    ix = idxs[:, 0]
    cross = n[:, None] * d[None, :]
    strictly = cross.T > cross
    tie = (cross.T == cross) & (ix[:, None] > ix[None, :])
    loses = jnp.any(strictly | tie, axis=1)
    best = jnp.min(jnp.where(loses, jnp.int32(V), ix))
    return best.astype(jnp.int32)
